# baseline (speedup 1.0000x reference)
import functools

import jax
import jax.numpy as jnp
from jax import lax
from jax.experimental import pallas as pl
from jax.experimental.pallas import tpu as pltpu
from jax.experimental.pallas import tpu_sc as plsc

NC = 2
NS = 16
NW = NC * NS

LANES = 16

AW = 512
ABLOCKS = 1952
ABLK_PER_W = ABLOCKS // NW
ATAILP = 144
APW = AW // 4

C = 256
GCHUNK = 128
NTASK_PER_W = 26 * 16384 // C // NW


def _format_table(tbl_t, tail):
    D, V = tbl_t.shape

    mesh = plsc.VectorSubcoreMesh(core_axis_name="c", subcore_axis_name="s")

    @functools.partial(
        pl.kernel,
        mesh=mesh,
        out_type=jax.ShapeDtypeStruct((V // 4, 128), jnp.float32),
        scratch_types=[
            pltpu.VMEM((2, D, AW), jnp.float32),
            pltpu.VMEM((2, APW, 128), jnp.float32),
            pltpu.VMEM((8, LANES), jnp.int32),
            pltpu.SemaphoreType.DMA,
            pltpu.SemaphoreType.DMA,
            pltpu.SemaphoreType.DMA,
            pltpu.SemaphoreType.DMA,
        ],
        compiler_params=pltpu.CompilerParams(needs_layout_passes=False),
    )
    def ka(tbl_hbm, tail_hbm, out_hbm, in_v, out_v, base_v, g0, g1, w0, w1):
        wid = lax.axis_index("s") * NC + lax.axis_index("c")
        gsems = (g0, g1)
        wsems = (w0, w1)

        @pl.when(wid == 0)
        def _():
            pltpu.sync_copy(tail_hbm,
                            out_hbm.at[pl.ds(ABLOCKS * APW, ATAILP)])

        for l0 in range(0, 128, LANES):
            base_v[l0 // LANES] = lax.iota(jnp.int32, LANES) + (l0 % 32)

        def fire_in(k, par):
            c0 = pl.multiple_of((wid * ABLK_PER_W + k) * AW, AW)
            for tr in range(D // 8):
                pltpu.async_copy(
                    tbl_hbm.at[pl.ds(tr * 8, 8), pl.ds(c0, AW)],
                    in_v.at[par, pl.ds(tr * 8, 8), :],
                    gsems[par],
                )

        def drain_in(par):
            for tr in range(D // 8):
                pltpu.make_async_copy(
                    tbl_hbm.at[pl.ds(0, 8), pl.ds(0, AW)],
                    in_v.at[par, pl.ds(tr * 8, 8), :],
                    gsems[par],
                ).wait()

        def step(k, par):
            @pl.when(k + 1 < ABLK_PER_W)
            def _():
                fire_in(k + 1, 1 - par)

            drain_in(par)

            @pl.when(k >= 2)
            def _():
                pltpu.make_async_copy(
                    out_v.at[par],
                    out_hbm.at[pl.ds(0, APW)],
                    wsems[par],
                ).wait()

            inb = in_v.at[par]
            outb = out_v.at[par]

            def row(r, _):
                r4v = lax.iota(jnp.int32, LANES) * 0 + (r * 4)
                for i in range(8):
                    outb[r, pl.ds(i * LANES, LANES)] = plsc.load_gather(
                        inb, [base_v[i], r4v + (i // 2)])
                return ()

            lax.fori_loop(0, APW, row, (), unroll=False)
            p0 = pl.multiple_of((wid * ABLK_PER_W + k) * APW, APW)
            pltpu.async_copy(out_v.at[par], out_hbm.at[pl.ds(p0, APW)],
                             wsems[par])

        fire_in(0, 0)

        def pair(i, _):
            step(i * 2, 0)
            step(i * 2 + 1, 1)
            return ()

        lax.fori_loop(0, ABLK_PER_W // 2, pair, (), unroll=False)
        step(ABLK_PER_W - 1, 0)
        for par in range(2):
            pltpu.make_async_copy(
                out_v.at[par], out_hbm.at[pl.ds(0, APW)], wsems[par]).wait()

    return ka(tbl_t, tail)


def _gather_tasks(ids_t, tbl2):
    F, S = ids_t.shape
    D = 32
    chunks_per_row = S // C
    N = NTASK_PER_W

    mesh = plsc.VectorSubcoreMesh(core_axis_name="c", subcore_axis_name="s")

    @functools.partial(
        pl.kernel,
        mesh=mesh,
        out_type=jax.ShapeDtypeStruct((F, D, S), jnp.float32),
        scratch_types=[
            pltpu.VMEM((2, 1, C), jnp.int32),
            pltpu.VMEM((2, C), jnp.int32),
            pltpu.VMEM((2, C), jnp.int32),
            pltpu.VMEM((2, C, 128), jnp.float32),
            pltpu.VMEM((2, 1, D, C), jnp.float32),
            pltpu.SemaphoreType.DMA,
            pltpu.SemaphoreType.DMA,
            pltpu.SemaphoreType.DMA,
            pltpu.SemaphoreType.DMA,
            pltpu.SemaphoreType.DMA,
            pltpu.SemaphoreType.DMA,
        ],
        compiler_params=pltpu.CompilerParams(needs_layout_passes=False),
    )
    def kb(ids_hbm, tbl_hbm, out_hbm, idr_v, idq_v, off_v, rows_v, out_v,
           i0, i1, g0, g1, w0, w1):
        wid = lax.axis_index("s") * NC + lax.axis_index("c")
        isems = (i0, i1)
        gsems = (g0, g1)
        wsems = (w0, w1)

        def taskpos(k):
            t = wid * N + k
            return t // chunks_per_row, (t % chunks_per_row) * C

        def fire_ids(k, par):
            f, s0 = taskpos(k)
            pltpu.async_copy(ids_hbm.at[pl.ds(f, 1), pl.ds(s0, C)],
                             idr_v.at[par], isems[par])

        def prep_and_fire_gathers(k, par):
            pltpu.make_async_copy(ids_hbm.at[pl.ds(0, 1), pl.ds(0, C)],
                                  idr_v.at[par], isems[par]).wait()
            for i in range(C // LANES):
                v = idr_v[par, 0, pl.ds(i * LANES, LANES)]
                off_v[par, pl.ds(i * LANES, LANES)] = lax.shift_left(
                    lax.bitwise_and(v, 3), 5)
                idq_v[par, pl.ds(i * LANES, LANES)] = (
                    lax.shift_right_logical(v, 2))
            for j in range(C // GCHUNK):
                pltpu.async_copy(
                    tbl_hbm.at[idq_v.at[par, pl.ds(j * GCHUNK, GCHUNK)]],
                    rows_v.at[par, pl.ds(j * GCHUNK, GCHUNK)],
                    gsems[par],
                )

        def step(k, par):
            @pl.when(k + 1 < N)
            def _():
                prep_and_fire_gathers(k + 1, 1 - par)

            @pl.when(k + 2 < N)
            def _():
                fire_ids(k + 2, par)

            for j in range(C // GCHUNK):
                pltpu.make_async_copy(
                    tbl_hbm.at[pl.ds(0, GCHUNK)],
                    rows_v.at[par, pl.ds(j * GCHUNK, GCHUNK)],
                    gsems[par],
                ).wait()

            @pl.when(k >= 2)
            def _():
                f0, s00 = taskpos(k - 2)
                pltpu.make_async_copy(
                    out_v.at[par],
                    out_hbm.at[pl.ds(f0, 1), :, pl.ds(s00, C)],
                    wsems[par],
                ).wait()

            rows = rows_v.at[par]
            outb = out_v.at[par, 0]

            def chunk(ji, _):
                j0 = ji * LANES
                rowi = lax.iota(jnp.int32, LANES) + j0
                colb = off_v[par, pl.ds(j0, LANES)]
                for d in range(D):
                    outb[d, pl.ds(j0, LANES)] = plsc.load_gather(
                        rows, [rowi, colb + d])
                return ()

            lax.fori_loop(0, C // LANES, chunk, (), unroll=False)
            f, s0 = taskpos(k)
            pltpu.async_copy(out_v.at[par],
                             out_hbm.at[pl.ds(f, 1), :, pl.ds(s0, C)],
                             wsems[par])

        fire_ids(0, 0)
        fire_ids(1, 1)
        prep_and_fire_gathers(0, 0)

        def pair(i, _):
            step(i * 2, 0)
            step(i * 2 + 1, 1)
            return ()

        lax.fori_loop(0, N // 2, pair, (), unroll=False)
        for par in range(2):
            f, s0 = taskpos(N - 2 + par)
            pltpu.make_async_copy(
                out_v.at[par], out_hbm.at[pl.ds(f, 1), :, pl.ds(s0, C)],
                wsems[par]).wait()

    return kb(ids_t, tbl2)


def kernel(input_ids, table):
    ids_t = input_ids.T.astype(jnp.int32)
    tail = table[ABLOCKS * AW:].reshape(ATAILP, 128)
    tbl2 = _format_table(table.T, tail)
    out3 = _gather_tasks(ids_t, tbl2)
    return jnp.transpose(out3, (2, 0, 1))


# speedup vs baseline: 2.0772x; 2.0772x over previous
import functools

import jax
import jax.numpy as jnp
from jax import lax
from jax.experimental import pallas as pl
from jax.experimental.pallas import tpu as pltpu
from jax.experimental.pallas import tpu_sc as plsc

NC = 2
NS = 16
NW = NC * NS

LANES = 16

AW = 512
ABLOCKS = 1952
ABLK_PER_W = ABLOCKS // NW
ATAILP = 144
APW = AW // 4

C = 256
GCHUNK = 128
NTASK_PER_W = 26 * 16384 // C // NW


def _format_table(tbl_t, tail):
    D, V = tbl_t.shape

    mesh = plsc.VectorSubcoreMesh(core_axis_name="c", subcore_axis_name="s")

    @functools.partial(
        pl.kernel,
        mesh=mesh,
        out_type=jax.ShapeDtypeStruct((V // 4, 128), jnp.float32),
        scratch_types=[
            pltpu.VMEM((2, D, AW), jnp.float32),
            pltpu.VMEM((2, APW, 128), jnp.float32),
            pltpu.VMEM((8, LANES), jnp.int32),
            pltpu.SemaphoreType.DMA,
            pltpu.SemaphoreType.DMA,
            pltpu.SemaphoreType.DMA,
            pltpu.SemaphoreType.DMA,
        ],
        compiler_params=pltpu.CompilerParams(needs_layout_passes=False),
    )
    def ka(tbl_hbm, tail_hbm, out_hbm, in_v, out_v, base_v, g0, g1, w0, w1):
        wid = lax.axis_index("s") * NC + lax.axis_index("c")
        gsems = (g0, g1)
        wsems = (w0, w1)

        @pl.when(wid == 0)
        def _():
            pltpu.sync_copy(tail_hbm,
                            out_hbm.at[pl.ds(ABLOCKS * APW, ATAILP)])

        for l0 in range(0, 128, LANES):
            base_v[l0 // LANES] = lax.iota(jnp.int32, LANES) + (l0 % 32)

        def fire_in(k, par):
            c0 = pl.multiple_of((wid * ABLK_PER_W + k) * AW, AW)
            for tr in range(D // 8):
                pltpu.async_copy(
                    tbl_hbm.at[pl.ds(tr * 8, 8), pl.ds(c0, AW)],
                    in_v.at[par, pl.ds(tr * 8, 8), :],
                    gsems[par],
                )

        def drain_in(par):
            for tr in range(D // 8):
                pltpu.make_async_copy(
                    tbl_hbm.at[pl.ds(0, 8), pl.ds(0, AW)],
                    in_v.at[par, pl.ds(tr * 8, 8), :],
                    gsems[par],
                ).wait()

        def step(k, par):
            @pl.when(k + 1 < ABLK_PER_W)
            def _():
                fire_in(k + 1, 1 - par)

            drain_in(par)

            @pl.when(k >= 2)
            def _():
                pltpu.make_async_copy(
                    out_v.at[par],
                    out_hbm.at[pl.ds(0, APW)],
                    wsems[par],
                ).wait()

            inb = in_v.at[par]
            outb = out_v.at[par]

            def row(r, _):
                r4v = lax.iota(jnp.int32, LANES) * 0 + (r * 4)
                for i in range(8):
                    outb[r, pl.ds(i * LANES, LANES)] = plsc.load_gather(
                        inb, [base_v[i], r4v + (i // 2)])
                return ()

            lax.fori_loop(0, APW, row, (), unroll=False)
            p0 = pl.multiple_of((wid * ABLK_PER_W + k) * APW, APW)
            pltpu.async_copy(out_v.at[par], out_hbm.at[pl.ds(p0, APW)],
                             wsems[par])

        fire_in(0, 0)

        def pair(i, _):
            step(i * 2, 0)
            step(i * 2 + 1, 1)
            return ()

        lax.fori_loop(0, ABLK_PER_W // 2, pair, (), unroll=False)
        step(ABLK_PER_W - 1, 0)
        for par in range(2):
            pltpu.make_async_copy(
                out_v.at[par], out_hbm.at[pl.ds(0, APW)], wsems[par]).wait()

    return ka(tbl_t, tail)


def _gather_tasks(ids_t, tbl2):
    F, S = ids_t.shape
    D = 32
    chunks_per_row = S // C
    N = NTASK_PER_W

    mesh = plsc.VectorSubcoreMesh(core_axis_name="c", subcore_axis_name="s")

    @functools.partial(
        pl.kernel,
        mesh=mesh,
        out_type=jax.ShapeDtypeStruct((F, D, S), jnp.float32),
        scratch_types=[
            pltpu.VMEM((2, 1, C), jnp.int32),
            pltpu.VMEM((2, C), jnp.int32),
            pltpu.VMEM((2, C), jnp.int32),
            pltpu.VMEM((2, C, 128), jnp.float32),
            pltpu.VMEM((2, 1, D, C), jnp.float32),
            pltpu.SemaphoreType.DMA,
            pltpu.SemaphoreType.DMA,
            pltpu.SemaphoreType.DMA,
            pltpu.SemaphoreType.DMA,
            pltpu.SemaphoreType.DMA,
            pltpu.SemaphoreType.DMA,
        ],
        compiler_params=pltpu.CompilerParams(needs_layout_passes=False),
    )
    def kb(ids_hbm, tbl_hbm, out_hbm, idr_v, idq_v, off_v, rows_v, out_v,
           i0, i1, g0, g1, w0, w1):
        wid = lax.axis_index("s") * NC + lax.axis_index("c")
        isems = (i0, i1)
        gsems = (g0, g1)
        wsems = (w0, w1)

        def taskpos(k):
            t = wid * N + k
            return t // chunks_per_row, (t % chunks_per_row) * C

        def fire_ids(k, par):
            f, s0 = taskpos(k)
            pltpu.async_copy(ids_hbm.at[pl.ds(f, 1), pl.ds(s0, C)],
                             idr_v.at[par], isems[par])

        def prep_and_fire_gathers(k, par):
            pltpu.make_async_copy(ids_hbm.at[pl.ds(0, 1), pl.ds(0, C)],
                                  idr_v.at[par], isems[par]).wait()
            for i in range(C // LANES):
                v = idr_v[par, 0, pl.ds(i * LANES, LANES)]
                off_v[par, pl.ds(i * LANES, LANES)] = lax.shift_left(
                    lax.bitwise_and(v, 3), 5)
                idq_v[par, pl.ds(i * LANES, LANES)] = (
                    lax.shift_right_logical(v, 2))
            for j in range(C // GCHUNK):
                pltpu.async_copy(
                    tbl_hbm.at[idq_v.at[par, pl.ds(j * GCHUNK, GCHUNK)]],
                    rows_v.at[par, pl.ds(j * GCHUNK, GCHUNK)],
                    gsems[par],
                )

        def step(k, par):
            @pl.when(k + 1 < N)
            def _():
                prep_and_fire_gathers(k + 1, 1 - par)

            @pl.when(k + 2 < N)
            def _():
                fire_ids(k + 2, par)

            for j in range(C // GCHUNK):
                pltpu.make_async_copy(
                    tbl_hbm.at[pl.ds(0, GCHUNK)],
                    rows_v.at[par, pl.ds(j * GCHUNK, GCHUNK)],
                    gsems[par],
                ).wait()

            @pl.when(k >= 2)
            def _():
                f0, s00 = taskpos(k - 2)
                pltpu.make_async_copy(
                    out_v.at[par],
                    out_hbm.at[pl.ds(f0, 1), :, pl.ds(s00, C)],
                    wsems[par],
                ).wait()

            rows = rows_v.at[par]
            outb = out_v.at[par, 0]

            def chunk(ji, _):
                j0 = ji * LANES
                rowi = lax.iota(jnp.int32, LANES) + j0
                colb = off_v[par, pl.ds(j0, LANES)]
                for d in range(D):
                    outb[d, pl.ds(j0, LANES)] = plsc.load_gather(
                        rows, [rowi, colb + d])
                return ()

            lax.fori_loop(0, C // LANES, chunk, (), unroll=False)
            f, s0 = taskpos(k)
            pltpu.async_copy(out_v.at[par],
                             out_hbm.at[pl.ds(f, 1), :, pl.ds(s0, C)],
                             wsems[par])

        fire_ids(0, 0)
        fire_ids(1, 1)
        prep_and_fire_gathers(0, 0)

        def pair(i, _):
            step(i * 2, 0)
            step(i * 2 + 1, 1)
            return ()

        lax.fori_loop(0, N // 2, pair, (), unroll=False)
        for par in range(2):
            f, s0 = taskpos(N - 2 + par)
            pltpu.make_async_copy(
                out_v.at[par], out_hbm.at[pl.ds(f, 1), :, pl.ds(s0, C)],
                wsems[par]).wait()

    return kb(ids_t, tbl2)


def kernel(input_ids, table):
    ids_t = input_ids.T.astype(jnp.int32)
    tbl2 = table.reshape(-1, 128)
    out3 = _gather_tasks(ids_t, tbl2)
    return jnp.transpose(out3, (2, 0, 1))


# speedup vs baseline: 2.1974x; 1.0578x over previous
import functools

import jax
import jax.numpy as jnp
from jax import lax
from jax.experimental import pallas as pl
from jax.experimental.pallas import tpu as pltpu
from jax.experimental.pallas import tpu_sc as plsc

NC = 2
NS = 16
NW = NC * NS

CHUNK = 128
CHUNKS_PER_BLK = 13
BLK = CHUNK * CHUNKS_PER_BLK


def _gather_rows(idx2, table):
    n_chunks = idx2.shape[0]
    D = table.shape[1]
    B = n_chunks * CHUNK
    chunks_per_w = n_chunks // NW
    n_blocks = chunks_per_w // CHUNKS_PER_BLK

    mesh = plsc.VectorSubcoreMesh(core_axis_name="c", subcore_axis_name="s")

    @functools.partial(
        pl.kernel,
        mesh=mesh,
        out_type=jax.ShapeDtypeStruct((B, D), jnp.float32),
        scratch_types=[
            pltpu.VMEM((chunks_per_w, CHUNK), jnp.int32),
            pltpu.VMEM((BLK, D), jnp.float32),
            pltpu.SemaphoreType.DMA,
        ],
        compiler_params=pltpu.CompilerParams(use_tc_tiling_on_sc=False),
    )
    def k(idx_hbm, table_hbm, out_hbm, idx_v, rows_v, sem):
        wid = lax.axis_index("s") * NC + lax.axis_index("c")
        chunk_base = wid * chunks_per_w
        out_base = chunk_base * CHUNK
        pltpu.sync_copy(idx_hbm.at[pl.ds(chunk_base, chunks_per_w)], idx_v)

        def body(blk, _):
            copies = []
            for j in range(CHUNKS_PER_BLK):
                cj = blk * CHUNKS_PER_BLK + j
                copies.append(
                    pltpu.async_copy(
                        table_hbm.at[idx_v.at[cj]],
                        rows_v.at[pl.ds(j * CHUNK, CHUNK)],
                        sem,
                    )
                )
            for c in copies:
                c.wait()
            pltpu.sync_copy(rows_v, out_hbm.at[pl.ds(out_base + blk * BLK, BLK)])
            return ()

        lax.fori_loop(0, n_blocks, body, (), unroll=False)

    return k(idx2, table)


def kernel(input_ids, table):
    S, F = input_ids.shape
    D = table.shape[1]
    idx = input_ids.reshape(-1).astype(jnp.int32)
    idx2 = idx.reshape(-1, CHUNK)
    out = _gather_rows(idx2, table)
    return out.reshape(S, F, D)
